# baseline (speedup 1.0000x reference)
import functools

import jax
import jax.numpy as jnp
from jax import lax
from jax.experimental import pallas as pl
from jax.experimental.pallas import tpu as pltpu
from jax.experimental.pallas import tpu_sc as plsc

_N_EXPERTS = 8
_TOP_K = 2
_AUX_COEF = 0.01
_LANES = 128
_TILE = 4096
_NW = 32
_L = 16


def _logits_body(h_ref, w_ref, o_ref):
    logits_t = jax.lax.dot_general(
        w_ref[...], h_ref[...],
        dimension_numbers=(((1,), (1,)), ((), ())),
        preferred_element_type=jnp.float32)
    o_ref[...] = logits_t[:_N_EXPERTS, :]


def _stats_body(pc_ref, aux_ref, load_ref, n_total):
    rows = jnp.sum(pc_ref[...], axis=0, keepdims=True)
    lr = jax.lax.broadcasted_iota(jnp.int32, rows.shape, 1)
    cnt = jnp.zeros_like(rows)
    for e in range(_N_EXPERTS):
        blk = jnp.logical_and(lr >= e * _L, lr < (e + 1) * _L)
        tot = jnp.sum(jnp.where(blk, rows, 0.0))
        cnt = cnt + jnp.where(lr == e, tot, 0.0)
    load = cnt / jnp.float32(n_total * _TOP_K)
    lvalid = lr < _N_EXPERTS
    mean = jnp.sum(jnp.where(lvalid, load, 0.0)) / _N_EXPERTS
    var = jnp.sum(jnp.where(lvalid, (load - mean) ** 2, 0.0)) / _N_EXPERTS
    cv_sq = var / (mean * mean + 1e-9)
    aux_ref[...] = jnp.full(aux_ref.shape, _AUX_COEF * cv_sq, jnp.float32)
    load_ref[...] = load


def _sc_router(logits_flat, n):
    chunk = n // _NW
    groups = chunk // _L
    mesh = plsc.VectorSubcoreMesh(core_axis_name="c", subcore_axis_name="s")

    @functools.partial(
        pl.kernel,
        out_type=(
            jax.ShapeDtypeStruct((n * _TOP_K,), jnp.int32),
            jax.ShapeDtypeStruct((n * _TOP_K,), jnp.float32),
            jax.ShapeDtypeStruct((_NW, _LANES), jnp.float32),
        ),
        mesh=mesh,
        scratch_types=[
            pltpu.VMEM((chunk * _N_EXPERTS,), jnp.float32),
            pltpu.VMEM((chunk * _TOP_K,), jnp.int32),
            pltpu.VMEM((chunk * _TOP_K,), jnp.float32),
            pltpu.VMEM((_LANES,), jnp.float32),
        ],
    )
    def sc_router(lg_hbm, idx_hbm, prob_hbm, pc_hbm,
                  lg_v, idx_v, prob_v, cnt_v):
        wid = lax.axis_index("s") * 2 + lax.axis_index("c")
        base = wid * chunk
        for e in range(_N_EXPERTS):
            pltpu.sync_copy(lg_hbm.at[pl.ds(e * n + base, chunk)],
                            lg_v.at[pl.ds(e * chunk, chunk)])
        iota = lax.iota(jnp.int32, _L)
        zero16 = jnp.zeros((_L,), jnp.float32)
        neg_big = jnp.float32(-3.0e38)
        pair_lo = iota >> 1
        pair_hi = pair_lo + (_L // 2)
        even = (iota & 1) == 0

        def body(g, accs):
            off = g * _L
            ls = [lg_v[pl.ds(e * chunk + off, _L)]
                  for e in range(_N_EXPERTS)]
            m = ls[0]
            for e in range(1, _N_EXPERTS):
                m = jnp.maximum(m, ls[e])
            i1 = jnp.full((_L,), _N_EXPERTS, jnp.int32)
            for e in range(_N_EXPERTS - 1, -1, -1):
                i1 = jnp.where(ls[e] == m, jnp.int32(e), i1)
            v2 = jnp.full((_L,), neg_big, jnp.float32)
            for e in range(_N_EXPERTS):
                v2 = jnp.maximum(v2, jnp.where(i1 == e, neg_big, ls[e]))
            i2 = jnp.full((_L,), _N_EXPERTS, jnp.int32)
            for e in range(_N_EXPERTS - 1, -1, -1):
                hit2 = jnp.logical_and(ls[e] == v2, i1 != e)
                i2 = jnp.where(hit2, jnp.int32(e), i2)
            s = zero16
            for e in range(_N_EXPERTS):
                s = s + jnp.exp(ls[e] - m)
            p1 = 1.0 / s
            p2 = jnp.exp(v2 - m) / s

            def take16(a, pp):
                return lax.gather(
                    a, pp[:, None],
                    dimension_numbers=lax.GatherDimensionNumbers(
                        offset_dims=(), collapsed_slice_dims=(0,),
                        start_index_map=(0,)),
                    slice_sizes=(1,),
                    mode=lax.GatherScatterMode.PROMISE_IN_BOUNDS)

            def ileave(a, b, pp):
                return jnp.where(even, take16(a, pp), take16(b, pp))

            idx_v[pl.ds(_TOP_K * off, _L)] = ileave(i1, i2, pair_lo)
            idx_v[pl.ds(_TOP_K * off + _L, _L)] = ileave(i1, i2, pair_hi)
            prob_v[pl.ds(_TOP_K * off, _L)] = ileave(p1, p2, pair_lo)
            prob_v[pl.ds(_TOP_K * off + _L, _L)] = ileave(p1, p2, pair_hi)
            new_accs = []
            for e in range(_N_EXPERTS):
                hit = jnp.logical_or(i1 == e, i2 == e)
                new_accs.append(accs[e] + jnp.where(hit, 1.0, 0.0))
            return tuple(new_accs)

        accs = lax.fori_loop(0, groups, body,
                             tuple(zero16 for _ in range(_N_EXPERTS)))
        for e in range(_N_EXPERTS):
            cnt_v[pl.ds(e * _L, _L)] = accs[e]
        pltpu.sync_copy(idx_v, idx_hbm.at[pl.ds(_TOP_K * base,
                                                 _TOP_K * chunk)])
        pltpu.sync_copy(prob_v, prob_hbm.at[pl.ds(_TOP_K * base,
                                                  _TOP_K * chunk)])
        pltpu.sync_copy(cnt_v, pc_hbm.at[wid])

    return sc_router(logits_flat)


def kernel(hidden_states, gate_weight):
    n, d = hidden_states.shape
    tile = _TILE
    logits_t = pl.pallas_call(
        _logits_body,
        grid=(n // tile,),
        in_specs=[
            pl.BlockSpec((tile, d), lambda i: (i, 0)),
            pl.BlockSpec((_LANES, d), lambda i: (0, 0)),
        ],
        out_specs=pl.BlockSpec((_N_EXPERTS, tile), lambda i: (0, i)),
        out_shape=jax.ShapeDtypeStruct((_N_EXPERTS, n), jnp.float32),
    )(hidden_states,
      jnp.pad(gate_weight, ((0, _LANES - _N_EXPERTS), (0, 0))))
    idx_flat, prob_flat, partials = _sc_router(logits_t.reshape(-1), n)
    aux, loadp = pl.pallas_call(
        functools.partial(_stats_body, n_total=n),
        out_shape=(
            jax.ShapeDtypeStruct((1, _LANES), jnp.float32),
            jax.ShapeDtypeStruct((1, _LANES), jnp.float32),
        ),
    )(partials)
    return (idx_flat.reshape(n, _TOP_K), prob_flat.reshape(n, _TOP_K),
            aux[0, 0], loadp[0, :_N_EXPERTS])


# speedup vs baseline: 1.7654x; 1.7654x over previous
import functools

import jax
import jax.numpy as jnp
from jax import lax
from jax.experimental import pallas as pl
from jax.experimental.pallas import tpu as pltpu
from jax.experimental.pallas import tpu_sc as plsc

_N_EXPERTS = 8
_TOP_K = 2
_AUX_COEF = 0.01
_LANES = 128
_TILE = 4096
_NW = 32
_L = 16


def _logits_body(h_ref, w_ref, o_ref):
    logits_t = jax.lax.dot_general(
        w_ref[...], h_ref[...],
        dimension_numbers=(((1,), (1,)), ((), ())),
        preferred_element_type=jnp.float32)
    o_ref[...] = logits_t[:_N_EXPERTS, :]


def _stats_body(pc_ref, aux_ref, load_ref, n_total):
    rows = jnp.sum(pc_ref[...], axis=0, keepdims=True)
    lr = jax.lax.broadcasted_iota(jnp.int32, rows.shape, 1)
    cnt = jnp.zeros_like(rows)
    for e in range(_N_EXPERTS):
        blk = jnp.logical_and(lr >= e * _L, lr < (e + 1) * _L)
        tot = jnp.sum(jnp.where(blk, rows, 0.0))
        cnt = cnt + jnp.where(lr == e, tot, 0.0)
    load = cnt / jnp.float32(n_total * _TOP_K)
    lvalid = lr < _N_EXPERTS
    mean = jnp.sum(jnp.where(lvalid, load, 0.0)) / _N_EXPERTS
    var = jnp.sum(jnp.where(lvalid, (load - mean) ** 2, 0.0)) / _N_EXPERTS
    cv_sq = var / (mean * mean + 1e-9)
    aux_ref[...] = jnp.full(aux_ref.shape, _AUX_COEF * cv_sq, jnp.float32)
    load_ref[...] = load


def _sc_router(logits_flat, n):
    chunk = n // _NW
    groups = chunk // _L
    mesh = plsc.VectorSubcoreMesh(core_axis_name="c", subcore_axis_name="s")

    @functools.partial(
        pl.kernel,
        out_type=(
            jax.ShapeDtypeStruct((n,), jnp.int32),
            jax.ShapeDtypeStruct((n,), jnp.int32),
            jax.ShapeDtypeStruct((n,), jnp.float32),
            jax.ShapeDtypeStruct((n,), jnp.float32),
            jax.ShapeDtypeStruct((_NW, _LANES), jnp.float32),
        ),
        mesh=mesh,
        scratch_types=[
            pltpu.VMEM((chunk * _N_EXPERTS,), jnp.float32),
            pltpu.VMEM((chunk,), jnp.int32),
            pltpu.VMEM((chunk,), jnp.int32),
            pltpu.VMEM((chunk,), jnp.float32),
            pltpu.VMEM((chunk,), jnp.float32),
            pltpu.VMEM((_LANES,), jnp.float32),
        ],
    )
    def sc_router(lg_hbm, i1_hbm, i2_hbm, p1_hbm, p2_hbm, pc_hbm,
                  lg_v, i1_v, i2_v, p1_v, p2_v, cnt_v):
        wid = lax.axis_index("s") * 2 + lax.axis_index("c")
        base = wid * chunk
        for e in range(_N_EXPERTS):
            pltpu.sync_copy(lg_hbm.at[pl.ds(e * n + base, chunk)],
                            lg_v.at[pl.ds(e * chunk, chunk)])
        iota = lax.iota(jnp.int32, _L)
        zero16 = jnp.zeros((_L,), jnp.float32)
        neg_big = jnp.float32(-3.0e38)

        def body(g, accs):
            off = g * _L
            ls = [lg_v[pl.ds(e * chunk + off, _L)]
                  for e in range(_N_EXPERTS)]
            m = ls[0]
            for e in range(1, _N_EXPERTS):
                m = jnp.maximum(m, ls[e])
            i1 = jnp.full((_L,), _N_EXPERTS, jnp.int32)
            for e in range(_N_EXPERTS - 1, -1, -1):
                i1 = jnp.where(ls[e] == m, jnp.int32(e), i1)
            v2 = jnp.full((_L,), neg_big, jnp.float32)
            for e in range(_N_EXPERTS):
                v2 = jnp.maximum(v2, jnp.where(i1 == e, neg_big, ls[e]))
            i2 = jnp.full((_L,), _N_EXPERTS, jnp.int32)
            for e in range(_N_EXPERTS - 1, -1, -1):
                hit2 = jnp.logical_and(ls[e] == v2, i1 != e)
                i2 = jnp.where(hit2, jnp.int32(e), i2)
            s = zero16
            for e in range(_N_EXPERTS):
                s = s + jnp.exp(ls[e] - m)
            i1_v[pl.ds(off, _L)] = i1
            i2_v[pl.ds(off, _L)] = i2
            p1_v[pl.ds(off, _L)] = 1.0 / s
            p2_v[pl.ds(off, _L)] = jnp.exp(v2 - m) / s
            new_accs = []
            for e in range(_N_EXPERTS):
                hit = jnp.logical_or(i1 == e, i2 == e)
                new_accs.append(accs[e] + jnp.where(hit, 1.0, 0.0))
            return tuple(new_accs)

        def body2(g2, accs):
            accs = body(g2 * 2, accs)
            return body(g2 * 2 + 1, accs)

        accs = lax.fori_loop(0, groups // 2, body2,
                             tuple(zero16 for _ in range(_N_EXPERTS)))
        for e in range(_N_EXPERTS):
            cnt_v[pl.ds(e * _L, _L)] = accs[e]
        pltpu.sync_copy(i1_v, i1_hbm.at[pl.ds(base, chunk)])
        pltpu.sync_copy(i2_v, i2_hbm.at[pl.ds(base, chunk)])
        pltpu.sync_copy(p1_v, p1_hbm.at[pl.ds(base, chunk)])
        pltpu.sync_copy(p2_v, p2_hbm.at[pl.ds(base, chunk)])
        pltpu.sync_copy(cnt_v, pc_hbm.at[wid])

    return sc_router(logits_flat)


def kernel(hidden_states, gate_weight):
    n, d = hidden_states.shape
    tile = _TILE
    wpad = jnp.pad(gate_weight, ((0, _LANES - _N_EXPERTS), (0, 0)))
    logits_t = pl.pallas_call(
        _logits_body,
        grid=(n // tile,),
        in_specs=[
            pl.BlockSpec((tile, d), lambda i: (i, 0)),
            pl.BlockSpec((_LANES, d), lambda i: (0, 0)),
        ],
        out_specs=pl.BlockSpec((_N_EXPERTS, tile), lambda i: (0, i)),
        out_shape=jax.ShapeDtypeStruct((_N_EXPERTS, n), jnp.float32),
    )(hidden_states, wpad)
    i1, i2, p1, p2, partials = _sc_router(logits_t.reshape(-1), n)
    aux, loadp = pl.pallas_call(
        functools.partial(_stats_body, n_total=n),
        out_shape=(
            jax.ShapeDtypeStruct((1, _LANES), jnp.float32),
            jax.ShapeDtypeStruct((1, _LANES), jnp.float32),
        ),
    )(partials)
    idx = jnp.stack([i1, i2], axis=1)
    prob = jnp.stack([p1, p2], axis=1)
    return (idx, prob, aux[0, 0], loadp[0, :_N_EXPERTS])
